# Initial kernel scaffold; baseline (speedup 1.0000x reference)
import jax
import jax.numpy as jnp
from jax.experimental import pallas as pl


def kernel(x, Wr, br, W1, b1, W2, b2):
    raise NotImplementedError("write your pallas kernel here")


# speedup vs baseline: 2.4061x; 2.4061x over previous
import functools

import jax
import jax.numpy as jnp
from jax import lax
from jax.experimental import pallas as pl
from jax.experimental.pallas import tpu as pltpu

T = 2048
H = 1024
E = 8
K = 2
FF = 4096
LBW = 0.01

TM = 1024
FC = 512
NT = T // TM
NF = FF // FC


def _moe_dense_body(x_ref, wr_ref, br_ref, w1_ref, b1_ref, w2_ref, b2_ref,
                    out_ref, loss_ref, w_scr, y_scr, psum_scr):
    t = pl.program_id(0)
    e = pl.program_id(1)
    f = pl.program_id(2)

    @pl.when((e == 0) & (f == 0))
    def _router():
        xb = x_ref[...]
        logits = jnp.dot(xb, wr_ref[...].T,
                         preferred_element_type=jnp.float32) + br_ref[...]
        m = jnp.max(logits, axis=-1, keepdims=True)
        p = jnp.exp(logits - m)
        p = p / jnp.sum(p, axis=-1, keepdims=True)

        colsum = jnp.sum(p, axis=0, keepdims=True)

        @pl.when(t == 0)
        def _():
            psum_scr[...] = colsum

        @pl.when(t != 0)
        def _():
            psum_scr[...] += colsum

        iota = lax.broadcasted_iota(jnp.int32, p.shape, 1)
        m1 = jnp.max(p, axis=-1, keepdims=True)
        i1 = jnp.min(jnp.where(p == m1, iota, E), axis=-1, keepdims=True)
        pm = jnp.where(iota == i1, -1.0, p)
        m2 = jnp.max(pm, axis=-1, keepdims=True)
        i2 = jnp.min(jnp.where(pm == m2, iota, E), axis=-1, keepdims=True)
        denom = m1 + m2 + 1e-8
        w_scr[...] = jnp.where(iota == i1, m1 / denom, 0.0) + \
                     jnp.where(iota == i2, m2 / denom, 0.0)

    xb = x_ref[...]
    h = jnp.dot(xb, w1_ref[0].T, preferred_element_type=jnp.float32)
    h = h + b1_ref[0, 0]
    h = 0.5 * h * (1.0 + lax.erf(h * 0.7071067811865476))
    yc = jnp.dot(h, w2_ref[0].T, preferred_element_type=jnp.float32)

    @pl.when(f == 0)
    def _():
        y_scr[...] = yc

    @pl.when(f != 0)
    def _():
        y_scr[...] += yc

    @pl.when(f == NF - 1)
    def _combine():
        eiota = lax.broadcasted_iota(jnp.int32, (1, E), 1)
        wcol = jnp.sum(jnp.where(eiota == e, w_scr[...], 0.0),
                       axis=-1, keepdims=True)
        contrib = (y_scr[...] + b2_ref[0]) * wcol

        @pl.when(e == 0)
        def _():
            out_ref[...] = contrib

        @pl.when(e != 0)
        def _():
            out_ref[...] += contrib

    @pl.when((t == NT - 1) & (e == E - 1) & (f == NF - 1))
    def _loss():
        avg = psum_scr[...] / T
        mean = jnp.sum(avg) / E
        var = jnp.sum((avg - mean) ** 2) / (E - 1)
        loss_ref[...] = jnp.broadcast_to(LBW * var, (1, 1))


@functools.partial(jax.jit, static_argnames=())
def _moe_dense(x2d, Wr, br2, W1, b1, W2, b2):
    grid = (NT, E, NF)
    out, loss = pl.pallas_call(
        _moe_dense_body,
        grid=grid,
        in_specs=[
            pl.BlockSpec((TM, H), lambda t, e, f: (t, 0)),
            pl.BlockSpec((E, H), lambda t, e, f: (0, 0)),
            pl.BlockSpec((1, E), lambda t, e, f: (0, 0)),
            pl.BlockSpec((1, FC, H), lambda t, e, f: (e, f, 0)),
            pl.BlockSpec((1, 1, 1, FC), lambda t, e, f: (e, f, 0, 0)),
            pl.BlockSpec((1, H, FC), lambda t, e, f: (e, 0, f)),
            pl.BlockSpec((1, 1, H), lambda t, e, f: (e, 0, 0)),
        ],
        out_specs=[
            pl.BlockSpec((TM, H), lambda t, e, f: (t, 0)),
            pl.BlockSpec((1, 1), lambda t, e, f: (0, 0)),
        ],
        out_shape=[
            jax.ShapeDtypeStruct((T, H), jnp.float32),
            jax.ShapeDtypeStruct((1, 1), jnp.float32),
        ],
        scratch_shapes=[
            pltpu.VMEM((TM, E), jnp.float32),
            pltpu.VMEM((TM, H), jnp.float32),
            pltpu.VMEM((1, E), jnp.float32),
        ],
    )(x2d, Wr, br2, W1, b1.reshape(E, NF, 1, FC), W2, b2.reshape(E, 1, H))
    return out, loss


def kernel(x, Wr, br, W1, b1, W2, b2):
    Bq, Sq, Hq = x.shape
    x2d = x.reshape(-1, Hq)
    br2 = br.reshape(1, E)
    out, loss = _moe_dense(x2d, Wr, br2, W1, b1, W2, b2)
    return out.reshape(Bq, Sq, Hq), loss.reshape(())
